# baseline (speedup 1.0000x reference)
import functools
import jax
import jax.numpy as jnp
from jax import lax
from jax.experimental import pallas as pl
from jax.experimental.pallas import tpu as pltpu
from jax.experimental.pallas import tpu_sc as plsc

_L = 16
_NW = 32
_R = 16
_NBUF = 6


def _tanh16(x):
    e = jnp.exp(x * 2.0)
    return 1.0 - 2.0 / (e + 1.0)


def _sc_kernel(n, d, nch):
    rows_per_w = n // _NW
    nchunk = rows_per_w // _R
    trips = nchunk // _NBUF
    kgrp = nch // _L

    mesh = plsc.VectorSubcoreMesh(core_axis_name="c", subcore_axis_name="s")

    @functools.partial(
        pl.kernel,
        out_type=jax.ShapeDtypeStruct((n, d), jnp.float32),
        mesh=mesh,
        scratch_types=[
            [pltpu.VMEM((_R, d), jnp.float32) for _ in range(_NBUF)],
            pltpu.VMEM((nch,), jnp.int32),
            [pltpu.SemaphoreType.DMA for _ in range(_NBUF)],
            [pltpu.SemaphoreType.DMA for _ in range(_NBUF)],
        ],
        compiler_params=pltpu.CompilerParams(needs_layout_passes=False),
    )
    def k(data_hbm, ch_hbm, out_hbm, bufs, ch_v, rsems, wsems):
        wid = lax.axis_index("s") * 2 + lax.axis_index("c")
        base = wid * rows_per_w
        pltpu.sync_copy(ch_hbm, ch_v)

        chv = [ch_v[pl.ds(g * _L, _L)] for g in range(kgrp)]

        def compute(buf):
            @plsc.parallel_loop(0, _R, unroll=2)
            def _rows(r):
                row = jnp.full((_L,), 0, jnp.int32) + r
                for g in range(kgrp):
                    x = plsc.load_gather(buf, [row, chv[g]])
                    plsc.store_scatter(buf, [row, chv[g]], _tanh16(x))

        def rd(g, j):
            return pltpu.make_async_copy(
                data_hbm.at[pl.ds(base + g * _R, _R)], bufs[j], rsems[j])

        def wr(g, j):
            return pltpu.make_async_copy(
                bufs[j], out_hbm.at[pl.ds(base + g * _R, _R)], wsems[j])

        for j in range(_NBUF):
            rd(j, j).start()

        def chunk_step(g, j):
            rd(g, j).wait()
            compute(bufs[j])
            wr(g, j).start()
            bpf = (j + 2) % _NBUF

            @pl.when(jnp.logical_and(g + 2 >= _NBUF, g + 2 < nchunk))
            def _prefetch():
                wr(g + 2 - _NBUF, bpf).wait()
                rd(g + 2, bpf).start()

        def trip(t, _):
            g0 = t * _NBUF
            for j in range(_NBUF):
                chunk_step(g0 + j, j)
            return 0

        full_trips = nchunk // _NBUF
        lax.fori_loop(0, full_trips, trip, 0)
        for g in range(full_trips * _NBUF, nchunk):
            chunk_step(g, g % _NBUF)
        for g in range(nchunk - _NBUF, nchunk):
            wr(g, g % _NBUF).wait()

    return k


def kernel(data, channels):
    n, d = data.shape
    return _sc_kernel(n, d, channels.shape[0])(data, channels)


# speedup vs baseline: 1.0391x; 1.0391x over previous
import functools
import jax
import jax.numpy as jnp
from jax import lax
from jax.experimental import pallas as pl
from jax.experimental.pallas import tpu as pltpu
from jax.experimental.pallas import tpu_sc as plsc

_L = 16
_NW = 32
_R = 32
_NBUF = 3


def _tanh16(x):
    e = jnp.exp(x * 2.0)
    return 1.0 - 2.0 / (e + 1.0)


def _sc_kernel(n, d, nch):
    rows_per_w = n // _NW
    nchunk = rows_per_w // _R
    trips = nchunk // _NBUF
    kgrp = nch // _L

    mesh = plsc.VectorSubcoreMesh(core_axis_name="c", subcore_axis_name="s")

    @functools.partial(
        pl.kernel,
        out_type=jax.ShapeDtypeStruct((n, d), jnp.float32),
        mesh=mesh,
        scratch_types=[
            [pltpu.VMEM((_R, d), jnp.float32) for _ in range(_NBUF)],
            pltpu.VMEM((nch,), jnp.int32),
            [pltpu.SemaphoreType.DMA for _ in range(_NBUF)],
            [pltpu.SemaphoreType.DMA for _ in range(_NBUF)],
        ],
        compiler_params=pltpu.CompilerParams(needs_layout_passes=False),
    )
    def k(data_hbm, ch_hbm, out_hbm, bufs, ch_v, rsems, wsems):
        wid = lax.axis_index("s") * 2 + lax.axis_index("c")
        base = wid * rows_per_w
        pltpu.sync_copy(ch_hbm, ch_v)

        chv = [ch_v[pl.ds(g * _L, _L)] for g in range(kgrp)]

        def compute(buf):
            @plsc.parallel_loop(0, _R, unroll=2)
            def _rows(r):
                row = jnp.full((_L,), 0, jnp.int32) + r
                for g in range(kgrp):
                    x = plsc.load_gather(buf, [row, chv[g]])
                    plsc.store_scatter(buf, [row, chv[g]], _tanh16(x))

        def rd(g, j):
            return pltpu.make_async_copy(
                data_hbm.at[pl.ds(base + g * _R, _R)], bufs[j], rsems[j])

        def wr(g, j):
            return pltpu.make_async_copy(
                bufs[j], out_hbm.at[pl.ds(base + g * _R, _R)], wsems[j])

        for j in range(_NBUF):
            rd(j, j).start()

        def chunk_step(g, j):
            rd(g, j).wait()
            compute(bufs[j])
            wr(g, j).start()
            bpf = (j + 2) % _NBUF

            @pl.when(jnp.logical_and(g + 2 >= _NBUF, g + 2 < nchunk))
            def _prefetch():
                wr(g + 2 - _NBUF, bpf).wait()
                rd(g + 2, bpf).start()

        def trip(t, _):
            g0 = t * _NBUF
            for j in range(_NBUF):
                chunk_step(g0 + j, j)
            return 0

        full_trips = nchunk // _NBUF
        lax.fori_loop(0, full_trips, trip, 0)
        for g in range(full_trips * _NBUF, nchunk):
            chunk_step(g, g % _NBUF)
        for g in range(nchunk - _NBUF, nchunk):
            wr(g, g % _NBUF).wait()

    return k


def kernel(data, channels):
    n, d = data.shape
    return _sc_kernel(n, d, channels.shape[0])(data, channels)
